# baseline (speedup 1.0000x reference)
import functools

import jax
import jax.numpy as jnp
from jax import lax
from jax.experimental import pallas as pl
from jax.experimental.pallas import tpu as pltpu
from jax.experimental.pallas import tpu_sc as plsc


def kernel(input_ids, token_table, pos_table):
    B, L = input_ids.shape
    V, D = token_table.shape
    LANES = 16
    HALF = L // 2

    info = plsc.get_sparse_core_info()
    NW = info.num_cores * info.num_subcores
    seqs_w = B // NW

    tab128 = jnp.pad(token_table, ((0, 0), (0, 128 - D)))

    ids2 = input_ids.reshape(B * 2, HALF)

    mesh = plsc.VectorSubcoreMesh(core_axis_name="c", subcore_axis_name="s")

    @functools.partial(
        pl.kernel,
        mesh=mesh,
        out_type=jax.ShapeDtypeStruct((B, L, D), jnp.float32),
        scratch_types=[
            pltpu.VMEM((L, D), jnp.float32),
            pltpu.VMEM((2 * seqs_w, HALF), jnp.int32),
            pltpu.VMEM((L, 128), jnp.float32),
            pltpu.VMEM((L, D), jnp.float32),
            pltpu.SemaphoreType.DMA,
        ],
    )
    def emb(ids_hbm, tab_hbm, pos_hbm, out_hbm, pos_v, idx_v, rows_v, out_v, sem):
        c = lax.axis_index("c")
        s = lax.axis_index("s")
        wid = s * info.num_cores + c
        base_seq = wid * seqs_w

        pltpu.sync_copy(pos_hbm, pos_v)
        pltpu.sync_copy(ids_hbm.at[pl.ds(base_seq * 2, 2 * seqs_w)], idx_v)

        def body(i, carry):
            cp0 = pltpu.async_copy(
                tab_hbm.at[idx_v.at[2 * i]], rows_v.at[pl.ds(0, HALF)], sem)
            cp1 = pltpu.async_copy(
                tab_hbm.at[idx_v.at[2 * i + 1]], rows_v.at[pl.ds(HALF, HALF)],
                sem)
            cp0.wait()
            cp1.wait()

            def add_row(r, carry2):
                for j in range(D // LANES):
                    sl = pl.ds(j * LANES, LANES)
                    out_v[r, sl] = rows_v[r, sl] + pos_v[r, sl]
                return carry2

            lax.fori_loop(0, L, add_row, 0)
            pltpu.sync_copy(out_v, out_hbm.at[base_seq + i])
            return carry

        lax.fori_loop(0, seqs_w, body, 0)

    return emb(ids2, tab128, pos_table)
